# baseline (speedup 1.0000x reference)
import functools

import jax
import jax.numpy as jnp
from jax import lax
from jax.experimental import pallas as pl
from jax.experimental.pallas import tpu as pltpu
from jax.experimental.pallas import tpu_sc as plsc

_NC = 2
_NS = 16
_NW = _NC * _NS
_L = 16

_D = 128
_CH = 80


def _proj_body(x_ref, wl_ref, bl_ref, wr_ref, br_ref, xl_ref, xr_ref):
    xb = x_ref[...]
    xl_ref[...] = jnp.dot(xb, wl_ref[...], preferred_element_type=jnp.float32) + bl_ref[...]
    xr_ref[...] = jnp.dot(xb, wr_ref[...], preferred_element_type=jnp.float32) + br_ref[...]


def _project(x, W_l, b_l, W_r, b_r):
    n, d = x.shape
    bn = 2000
    return pl.pallas_call(
        _proj_body,
        grid=(n // bn,),
        in_specs=[
            pl.BlockSpec((bn, d), lambda i: (i, 0)),
            pl.BlockSpec((d, d), lambda i: (0, 0)),
            pl.BlockSpec((1, d), lambda i: (0, 0)),
            pl.BlockSpec((d, d), lambda i: (0, 0)),
            pl.BlockSpec((1, d), lambda i: (0, 0)),
        ],
        out_specs=[
            pl.BlockSpec((bn, d), lambda i: (i, 0)),
            pl.BlockSpec((bn, d), lambda i: (i, 0)),
        ],
        out_shape=[
            jax.ShapeDtypeStruct((n, d), jnp.float32),
            jax.ShapeDtypeStruct((n, d), jnp.float32),
        ],
    )(x, W_l, b_l.reshape(1, d), W_r, b_r.reshape(1, d))


def _finalize_body(acc_ref, den_ref, bias_ref, out_ref):
    d = jnp.sum(den_ref[...], axis=0)
    a = acc_ref[0] + acc_ref[1]
    safe = jnp.where(d > 0, d, 1.0)
    out_ref[...] = a / safe[:, None] + bias_ref[...]


def _finalize(acc, den, bias):
    n = acc.shape[1]
    return pl.pallas_call(
        _finalize_body,
        out_shape=jax.ShapeDtypeStruct((n, _D), jnp.float32),
    )(acc, den, bias.reshape(1, _D))


def _sc_edge_pass(x_l, x_r, eidx_il, att):
    n = x_l.shape[0]
    e = eidx_il.shape[0] // 2
    assert e % _NW == 0
    per_tile = e // _NW
    assert per_tile % _CH == 0
    n_chunks = per_tile // _CH
    assert n_chunks >= 5 and (n_chunks - 1) % 4 == 0
    assert n % _CH == 0
    nzc = n // _CH
    zk = (nzc + _NS - 1) // _NS

    mesh = plsc.VectorSubcoreMesh(core_axis_name="c", subcore_axis_name="s",
                                  num_cores=_NC, num_subcores=_NS)

    @functools.partial(
        pl.kernel,
        out_type=[
            jax.ShapeDtypeStruct((_NC, n, _D), jnp.float32),
            jax.ShapeDtypeStruct((_NC * n,), jnp.float32),
        ],
        mesh=mesh,
        compiler_params=pltpu.CompilerParams(needs_layout_passes=False),
        scratch_types=[
            pltpu.VMEM((2 * _CH,), jnp.int32),
            pltpu.VMEM((2 * _CH,), jnp.int32),
            pltpu.VMEM((2 * _CH,), jnp.int32),
            pltpu.VMEM((2 * _CH,), jnp.int32),
            pltpu.VMEM((1, _CH), jnp.int32),
            pltpu.VMEM((1, _CH), jnp.int32),
            pltpu.VMEM((_CH, _D), jnp.float32),
            pltpu.VMEM((_CH, _D), jnp.float32),
            pltpu.VMEM((_CH, _D), jnp.float32),
            pltpu.VMEM((_CH, _D), jnp.float32),
            pltpu.VMEM((_CH,), jnp.float32),
            pltpu.VMEM((_CH,), jnp.float32),
            pltpu.VMEM((_D,), jnp.float32),
            pltpu.VMEM((_L * _L,), jnp.float32),
            pltpu.VMEM_SHARED((n, _D), jnp.float32),
            pltpu.VMEM_SHARED((n,), jnp.float32),
            pltpu.SemaphoreType.DMA,
            pltpu.SemaphoreType.DMA,
            pltpu.SemaphoreType.DMA,
            pltpu.SemaphoreType.DMA,
            pltpu.SemaphoreType.DMA,
            pltpu.SemaphoreType.DMA,
            pltpu.SemaphoreType.DMA,
        ],
    )
    def sc_kernel(xl_hbm, xr_hbm, il_hbm, att_hbm, acc_hbm, den_hbm,
                  il0, il1, il2, il3, sidxA, sidxB, xlA, xrA, xlB, xrB, pA, pB,
                  att_v, ebuf, acc_sh, den_sh, semA, semB,
                  isem0, isem1, isem2, isem3, ssem):
        cid = lax.axis_index("c")
        sid = lax.axis_index("s")
        wid = cid * _NS + sid
        base = wid * per_tile

        pltpu.sync_copy(att_hbm, att_v)

        z16 = jnp.zeros((_L,), jnp.float32)

        for q in range(_CH // _L):
            pA[pl.ds(q * _L, _L)] = z16

        def zrow(i, carry):
            xlA[i // (_D // _L), pl.ds((i % (_D // _L)) * _L, _L)] = z16
            return carry

        lax.fori_loop(0, _CH * (_D // _L), zrow, 0)

        def zacc(k, carry):
            c = sid + k * _NS

            @pl.when(c < nzc)
            def _():
                pltpu.sync_copy(xlA, acc_sh.at[pl.ds(c * _CH, _CH)])
                pltpu.sync_copy(pA, den_sh.at[pl.ds(c * _CH, _CH)])

            return carry

        lax.fori_loop(0, zk, zacc, 0)
        plsc.subcore_barrier()

        att_regs = [att_v[pl.ds(j * _L, _L)] for j in range(_D // _L)]
        lane = lax.broadcasted_iota(jnp.int32, (_L,), 0)
        lane16 = lane * _L

        ils = [(il0, isem0), (il1, isem1), (il2, isem2), (il3, isem3)]

        def idx_load(ci, I):
            il_v, isem = I
            pltpu.async_copy(
                il_hbm.at[pl.ds(2 * base + ci * 2 * _CH, 2 * _CH)], il_v, isem)

        def idx_wait(I):
            il_v, isem = I
            pltpu.make_async_copy(
                il_hbm.at[pl.ds(2 * base, 2 * _CH)], il_v, isem).wait()

        def gathers(I, xl_v, xr_v, sem):
            il_v, _ = I
            pltpu.async_copy(xl_hbm.at[il_v.at[pl.ds(0, _CH)]], xl_v, sem)
            pltpu.async_copy(xr_hbm.at[il_v.at[pl.ds(_CH, _CH)]], xr_v, sem)

        def wait_g(I, xl_v, xr_v, sem):
            il_v, _ = I
            pltpu.make_async_copy(xl_hbm.at[il_v.at[pl.ds(0, _CH)]], xl_v,
                                  sem).wait()
            pltpu.make_async_copy(xr_hbm.at[il_v.at[pl.ds(_CH, _CH)]], xr_v,
                                  sem).wait()

        def compute_scatter(I, sidx, xl_v, xr_v, p_v):
            il_v, _ = I
            def group(g, carry):
                for q in range(_L):
                    k = g * _L + q
                    acc = jnp.zeros((_L,), jnp.float32)
                    for j in range(_D // _L):
                        a = xl_v[k, pl.ds(j * _L, _L)]
                        b = xr_v[k, pl.ds(j * _L, _L)]
                        v = a + b
                        acc = acc + jnp.maximum(v, 0.2 * v) * att_regs[j]
                    ebuf[pl.ds(q * _L, _L)] = acc
                tot = jnp.zeros((_L,), jnp.float32)
                for l in range(_L):
                    tot = tot + plsc.load_gather(ebuf, [lane16 + l])
                pv16 = jnp.exp(tot)
                p_v[pl.ds(g * _L, _L)] = pv16
                for q in range(_L):
                    k = g * _L + q
                    ps = plsc.load_gather(p_v, [jnp.full((_L,), k, jnp.int32)])
                    for j in range(_D // _L):
                        xl_v[k, pl.ds(j * _L, _L)] = xl_v[k, pl.ds(j * _L, _L)] * ps
                return carry

            lax.fori_loop(0, _CH // _L, group, 0)
            for q in range(_CH // _L):
                sidx[0, pl.ds(q * _L, _L)] = il_v[pl.ds(_CH + q * _L, _L)]

        SA = (sidxA, xlA, xrA, pA, semA)
        SB = (sidxB, xlB, xrB, pB, semB)

        def scatter_issue(S):
            sidx, xl_v, _, p_v, _ = S
            pltpu.async_copy(xl_v, acc_sh.at[sidx.at[0]], ssem, add=True)
            pltpu.async_copy(p_v, den_sh.at[sidx.at[0]], ssem, add=True)

        def wait_s(S):
            sidx, xl_v, _, p_v, _ = S
            pltpu.make_async_copy(xl_v, acc_sh.at[sidx.at[0]], ssem).wait()
            pltpu.make_async_copy(p_v, den_sh.at[sidx.at[0]], ssem).wait()

        def chunk_step(c, I_cur, I_nxt, I_pre, S_cur, S_nxt, first=False):
            sidx, xl_v, xr_v, p_v, sem = S_cur
            _, xl_n, xr_n, _, sem_n = S_nxt
            wait_g(I_cur, xl_v, xr_v, sem)
            compute_scatter(I_cur, sidx, xl_v, xr_v, p_v)
            if not first:
                wait_s(S_nxt)

            @pl.when(c + 1 < n_chunks)
            def _():
                idx_wait(I_nxt)
                gathers(I_nxt, xl_n, xr_n, sem_n)

            scatter_issue(S_cur)

            @pl.when(c + 3 < n_chunks)
            def _():
                idx_load(c + 3, I_pre)

        IL0, IL1, IL2, IL3 = ils
        idx_load(0, IL0)
        idx_load(1, IL1)
        idx_load(2, IL2)
        idx_wait(IL0)
        gathers(IL0, xlA, xrA, semA)
        chunk_step(0, IL0, IL1, IL3, SA, SB, first=True)

        def quad(i, carry):
            c = 1 + 4 * i
            chunk_step(c, IL1, IL2, IL0, SB, SA)
            chunk_step(c + 1, IL2, IL3, IL1, SA, SB)
            chunk_step(c + 2, IL3, IL0, IL2, SB, SA)
            chunk_step(c + 3, IL0, IL1, IL3, SA, SB)
            return carry

        lax.fori_loop(0, (n_chunks - 1) // 4, quad, 0)
        wait_s(SA)

        plsc.subcore_barrier()

        def rdout(k, carry):
            c = sid + k * _NS

            @pl.when(c < nzc)
            def _():
                pltpu.sync_copy(acc_sh.at[pl.ds(c * _CH, _CH)],
                                acc_hbm.at[cid, pl.ds(c * _CH, _CH)])
                pltpu.sync_copy(den_sh.at[pl.ds(c * _CH, _CH)], pB)
                pltpu.sync_copy(pB, den_hbm.at[pl.ds(cid * n + c * _CH, _CH)])

            return carry

        lax.fori_loop(0, zk, rdout, 0)

    return sc_kernel(x_l, x_r, eidx_il, att)


def kernel(x, edge_index, valid_lens, time_step_len, W_l, b_l, W_r, b_r, att, bias):
    x_l, x_r = _project(x, W_l, b_l, W_r, b_r)
    eidx = edge_index.astype(jnp.int32)
    e = eidx.shape[1]
    eidx_il = eidx.reshape(2, e // _CH, _CH).transpose(1, 0, 2).reshape(-1)
    acc, den = _sc_edge_pass(x_l, x_r, eidx_il, att)
    return _finalize(acc, den.reshape(_NC, x.shape[0]), bias)
